# baseline (speedup 1.0000x reference)
import functools

import jax
import jax.numpy as jnp
from jax.experimental import pallas as pl

_N = 256
_CHUNK = 16


def _layer(xb, maskT, wl_ref, wr_ref, att_ref, b_ref):
    wl = wl_ref[...]
    wr = wr_ref[...]
    att = att_ref[...]
    h = wl.shape[1]

    xl = jnp.dot(xb, wl, preferred_element_type=jnp.float32)
    xr = jnp.dot(xb, wr, preferred_element_type=jnp.float32)

    csc = jnp.abs(att)
    sgn4 = jnp.where(att >= 0, 0.4, -0.4)
    xls = xl * csc
    xrs = xr * csc
    tl = 0.6 * jnp.sum(xl * att, axis=1, keepdims=True)
    tr = 0.6 * jnp.sum(xr * att, axis=1, keepdims=True)
    tl_row = tl.reshape(1, _N)

    neg = jnp.float32(-1e30)
    chunks = []
    for i in range(_N // _CHUNK):
        xrs_c = xrs[i * _CHUNK:(i + 1) * _CHUNK]
        u = xrs_c[:, None, :] + xls[None, :, :]
        t = jnp.abs(u) * sgn4[None, :, :]
        s = jnp.sum(t, axis=-1)
        s = s + tl_row + tr[i * _CHUNK:(i + 1) * _CHUNK]
        m = maskT[i * _CHUNK:(i + 1) * _CHUNK]
        s = jnp.where(m, s, neg)
        rmax = jnp.max(s, axis=1, keepdims=True)
        ea = jnp.where(m, jnp.exp(s - rmax), 0.0)
        den = jnp.sum(ea, axis=1, keepdims=True)
        chunks.append(ea / (den + 1e-16))
    p = jnp.concatenate(chunks, axis=0)
    out = jnp.dot(p, xl, preferred_element_type=jnp.float32)
    return out + b_ref[...]


def _gat3_kernel(x_ref, adjt_ref, wl0, wr0, at0, b0, wl1, wr1, at1, b1,
                 wl2, wr2, at2, b2, out_ref):
    maskT = adjt_ref[0] > 0
    xb = x_ref[0]
    xb = _layer(xb, maskT, wl0, wr0, at0, b0)
    xb = _layer(xb, maskT, wl1, wr1, at1, b1)
    xb = _layer(xb, maskT, wl2, wr2, at2, b2)
    out_ref[0] = xb


@jax.jit
def kernel(batch_graph, adj, Wl0, Wr0, att0, b0, Wl1, Wr1, att1, b1,
           Wl2, Wr2, att2, b2):
    bsz, n, _ = batch_graph.shape
    dout = Wl2.shape[1]
    adjt = adj.transpose(0, 2, 1)

    def wspec(w):
        return pl.BlockSpec(w.shape, lambda b: (0,) * w.ndim)

    weights = [Wl0, Wr0, att0.reshape(1, -1), b0.reshape(1, -1),
               Wl1, Wr1, att1.reshape(1, -1), b1.reshape(1, -1),
               Wl2, Wr2, att2.reshape(1, -1), b2.reshape(1, -1)]

    out = pl.pallas_call(
        _gat3_kernel,
        grid=(bsz,),
        in_specs=[
            pl.BlockSpec((1, n, batch_graph.shape[2]), lambda b: (b, 0, 0)),
            pl.BlockSpec((1, n, n), lambda b: (b, 0, 0)),
        ] + [wspec(w) for w in weights],
        out_specs=pl.BlockSpec((1, n, dout), lambda b: (b, 0, 0)),
        out_shape=jax.ShapeDtypeStruct((bsz, n, dout), jnp.float32),
    )(batch_graph, adjt, *weights)
    return out


# speedup vs baseline: 1.4771x; 1.4771x over previous
import functools

import jax
import jax.numpy as jnp
from jax.experimental import pallas as pl

_N = 256
_CHUNK = 16


def _layer(xb, maskT, wl_ref, wr_ref, att_ref, b_ref):
    wl = wl_ref[...]
    wr = wr_ref[...]
    att = att_ref[...]
    h = wl.shape[1]

    xl = jnp.dot(xb, wl, preferred_element_type=jnp.float32)
    xr = jnp.dot(xb, wr, preferred_element_type=jnp.float32)

    csc = jnp.abs(att)
    sgn = jnp.where(att >= 0, 1.0, -1.0).astype(jnp.float32)
    xls = xl * csc
    xrs = xr * csc

    neg = jnp.float32(-1e30)
    chunks = []
    for i in range(_N // _CHUNK):
        xrs_c = xrs[i * _CHUNK:(i + 1) * _CHUNK]
        u = xrs_c[:, None, :] + xls[None, :, :]
        t = jnp.maximum(u, 0.2 * u) * sgn[None, :, :]
        s = jnp.sum(t, axis=-1)
        m = maskT[i * _CHUNK:(i + 1) * _CHUNK]
        s = jnp.where(m, s, neg)
        rmax = jnp.max(s, axis=1, keepdims=True)
        ea = jnp.where(m, jnp.exp(s - rmax), 0.0)
        den = jnp.sum(ea, axis=1, keepdims=True)
        chunks.append(ea / (den + 1e-16))
    p = jnp.concatenate(chunks, axis=0)
    out = jnp.dot(p, xl, preferred_element_type=jnp.float32)
    return out + b_ref[...]


def _gat3_kernel(x_ref, adjt_ref, wl0, wr0, at0, b0, wl1, wr1, at1, b1,
                 wl2, wr2, at2, b2, out_ref):
    maskT = adjt_ref[0] > 0
    xb = x_ref[0]
    xb = _layer(xb, maskT, wl0, wr0, at0, b0)
    xb = _layer(xb, maskT, wl1, wr1, at1, b1)
    xb = _layer(xb, maskT, wl2, wr2, at2, b2)
    out_ref[0] = xb


@jax.jit
def kernel(batch_graph, adj, Wl0, Wr0, att0, b0, Wl1, Wr1, att1, b1,
           Wl2, Wr2, att2, b2):
    bsz, n, _ = batch_graph.shape
    dout = Wl2.shape[1]
    adjt = adj.transpose(0, 2, 1)

    def wspec(w):
        return pl.BlockSpec(w.shape, lambda b: (0,) * w.ndim)

    weights = [Wl0, Wr0, att0.reshape(1, -1), b0.reshape(1, -1),
               Wl1, Wr1, att1.reshape(1, -1), b1.reshape(1, -1),
               Wl2, Wr2, att2.reshape(1, -1), b2.reshape(1, -1)]

    out = pl.pallas_call(
        _gat3_kernel,
        grid=(bsz,),
        in_specs=[
            pl.BlockSpec((1, n, batch_graph.shape[2]), lambda b: (b, 0, 0)),
            pl.BlockSpec((1, n, n), lambda b: (b, 0, 0)),
        ] + [wspec(w) for w in weights],
        out_specs=pl.BlockSpec((1, n, dout), lambda b: (b, 0, 0)),
        out_shape=jax.ShapeDtypeStruct((bsz, n, dout), jnp.float32),
    )(batch_graph, adjt, *weights)
    return out


# speedup vs baseline: 2.2636x; 1.5325x over previous
import functools

import jax
import jax.numpy as jnp
from jax.experimental import pallas as pl

_N = 256
_CHUNK = 16


def _layer(xb, maskT, wl_ref, wr_ref, att_ref, b_ref):
    wl = wl_ref[...]
    wr = wr_ref[...]
    att = att_ref[...]
    h = wl.shape[1]

    xl = jnp.dot(xb, wl, preferred_element_type=jnp.float32)
    xr = jnp.dot(xb, wr, preferred_element_type=jnp.float32)

    csc = jnp.abs(att)
    sgn4 = jnp.where(att >= 0, 0.4, -0.4).astype(jnp.float32)
    xls = xl * csc
    xrs = xr * csc
    tl = 0.6 * jnp.sum(xl * att, axis=1, keepdims=True)
    gl = jnp.exp(tl - jnp.max(tl))

    neg = jnp.float32(-1e30)
    chunks = []
    for i in range(_N // _CHUNK):
        xrs_c = xrs[i * _CHUNK:(i + 1) * _CHUNK]
        u = xrs_c[:, None, :] + xls[None, :, :]
        t = jnp.abs(u) * sgn4[None, :, :]
        s = jnp.sum(t, axis=-1)
        m = maskT[i * _CHUNK:(i + 1) * _CHUNK]
        s = jnp.where(m, s, neg)
        rmax = jnp.max(s, axis=1, keepdims=True)
        chunks.append(jnp.where(m, jnp.exp(s - rmax), 0.0))
    w = jnp.concatenate(chunks, axis=0)
    num = jnp.dot(w, xl * gl, preferred_element_type=jnp.float32)
    den = jnp.dot(w, gl, preferred_element_type=jnp.float32)
    out = num / (den + 1e-16)
    return out + b_ref[...]


def _gat3_kernel(x_ref, adjt_ref, wl0, wr0, at0, b0, wl1, wr1, at1, b1,
                 wl2, wr2, at2, b2, out_ref):
    maskT = adjt_ref[0] > 0
    xb = x_ref[0]
    xb = _layer(xb, maskT, wl0, wr0, at0, b0)
    xb = _layer(xb, maskT, wl1, wr1, at1, b1)
    xb = _layer(xb, maskT, wl2, wr2, at2, b2)
    out_ref[0] = xb


@jax.jit
def kernel(batch_graph, adj, Wl0, Wr0, att0, b0, Wl1, Wr1, att1, b1,
           Wl2, Wr2, att2, b2):
    bsz, n, _ = batch_graph.shape
    dout = Wl2.shape[1]
    adjt = adj.transpose(0, 2, 1)

    def wspec(w):
        return pl.BlockSpec(w.shape, lambda b: (0,) * w.ndim)

    weights = [Wl0, Wr0, att0.reshape(1, -1), b0.reshape(1, -1),
               Wl1, Wr1, att1.reshape(1, -1), b1.reshape(1, -1),
               Wl2, Wr2, att2.reshape(1, -1), b2.reshape(1, -1)]

    out = pl.pallas_call(
        _gat3_kernel,
        grid=(bsz,),
        in_specs=[
            pl.BlockSpec((1, n, batch_graph.shape[2]), lambda b: (b, 0, 0)),
            pl.BlockSpec((1, n, n), lambda b: (b, 0, 0)),
        ] + [wspec(w) for w in weights],
        out_specs=pl.BlockSpec((1, n, dout), lambda b: (b, 0, 0)),
        out_shape=jax.ShapeDtypeStruct((bsz, n, dout), jnp.float32),
    )(batch_graph, adjt, *weights)
    return out


# speedup vs baseline: 2.3955x; 1.0583x over previous
import functools

import jax
import jax.numpy as jnp
from jax.experimental import pallas as pl

_N = 256
_CHUNK = 16


def _layer(xb, maskT, wl_ref, wr_ref, att_ref, b_ref):
    wl = wl_ref[...]
    wr = wr_ref[...]
    att = att_ref[...]
    h = wl.shape[1]

    xl = jnp.dot(xb, wl, preferred_element_type=jnp.float32)
    xr = jnp.dot(xb, wr, preferred_element_type=jnp.float32)

    csc = jnp.abs(att)
    sgn4 = jnp.where(att >= 0, 0.4, -0.4).astype(jnp.float32)
    xls_t = (xl * csc).T
    xrs = xr * csc
    tl = 0.6 * jnp.sum(xl * att, axis=1, keepdims=True)
    gl = jnp.exp(tl - jnp.max(tl))

    neg = jnp.float32(-1e30)
    chunks = []
    for i in range(_N // _CHUNK):
        xrs_c = xrs[i * _CHUNK:(i + 1) * _CHUNK]
        u = xrs_c[:, :, None] + xls_t[None, :, :]
        t = jnp.abs(u) * sgn4[0][None, :, None]
        s = jnp.sum(t, axis=1)
        m = maskT[i * _CHUNK:(i + 1) * _CHUNK]
        s = jnp.where(m, s, neg)
        rmax = jnp.max(s, axis=1, keepdims=True)
        chunks.append(jnp.where(m, jnp.exp(s - rmax), 0.0))
    w = jnp.concatenate(chunks, axis=0)
    num = jnp.dot(w, xl * gl, preferred_element_type=jnp.float32)
    den = jnp.dot(w, gl, preferred_element_type=jnp.float32)
    out = num / (den + 1e-16)
    return out + b_ref[...]


def _gat3_kernel(x_ref, adjt_ref, wl0, wr0, at0, b0, wl1, wr1, at1, b1,
                 wl2, wr2, at2, b2, out_ref):
    maskT = adjt_ref[0] > 0
    xb = x_ref[0]
    xb = _layer(xb, maskT, wl0, wr0, at0, b0)
    xb = _layer(xb, maskT, wl1, wr1, at1, b1)
    xb = _layer(xb, maskT, wl2, wr2, at2, b2)
    out_ref[0] = xb


@jax.jit
def kernel(batch_graph, adj, Wl0, Wr0, att0, b0, Wl1, Wr1, att1, b1,
           Wl2, Wr2, att2, b2):
    bsz, n, _ = batch_graph.shape
    dout = Wl2.shape[1]
    adjt = adj.transpose(0, 2, 1)

    def wspec(w):
        return pl.BlockSpec(w.shape, lambda b: (0,) * w.ndim)

    weights = [Wl0, Wr0, att0.reshape(1, -1), b0.reshape(1, -1),
               Wl1, Wr1, att1.reshape(1, -1), b1.reshape(1, -1),
               Wl2, Wr2, att2.reshape(1, -1), b2.reshape(1, -1)]

    out = pl.pallas_call(
        _gat3_kernel,
        grid=(bsz,),
        in_specs=[
            pl.BlockSpec((1, n, batch_graph.shape[2]), lambda b: (b, 0, 0)),
            pl.BlockSpec((1, n, n), lambda b: (b, 0, 0)),
        ] + [wspec(w) for w in weights],
        out_specs=pl.BlockSpec((1, n, dout), lambda b: (b, 0, 0)),
        out_shape=jax.ShapeDtypeStruct((bsz, n, dout), jnp.float32),
    )(batch_graph, adjt, *weights)
    return out


# speedup vs baseline: 2.4059x; 1.0044x over previous
import functools

import jax
import jax.numpy as jnp
from jax.experimental import pallas as pl

_N = 256
_CHUNK = 32


def _layer(xb, maskT, wl_ref, wr_ref, att_ref, b_ref):
    wl = wl_ref[...]
    wr = wr_ref[...]
    att = att_ref[...]
    h = wl.shape[1]

    xl = jnp.dot(xb, wl, preferred_element_type=jnp.float32)
    xr = jnp.dot(xb, wr, preferred_element_type=jnp.float32)

    csc = jnp.abs(att)
    sgn4 = jnp.where(att >= 0, 0.4, -0.4).astype(jnp.float32)
    xls_t = (xl * csc).T
    xrs = xr * csc
    tl = 0.6 * jnp.sum(xl * att, axis=1, keepdims=True)
    gl = jnp.exp(tl - jnp.max(tl))

    neg = jnp.float32(-1e30)
    chunks = []
    for i in range(_N // _CHUNK):
        xrs_c = xrs[i * _CHUNK:(i + 1) * _CHUNK]
        u = xrs_c[:, :, None] + xls_t[None, :, :]
        t = jnp.abs(u) * sgn4[0][None, :, None]
        s = jnp.sum(t, axis=1)
        m = maskT[i * _CHUNK:(i + 1) * _CHUNK]
        s = jnp.where(m, s, neg)
        rmax = jnp.max(s, axis=1, keepdims=True)
        chunks.append(jnp.where(m, jnp.exp(s - rmax), 0.0))
    w = jnp.concatenate(chunks, axis=0)
    num = jnp.dot(w, xl * gl, preferred_element_type=jnp.float32)
    den = jnp.dot(w, gl, preferred_element_type=jnp.float32)
    out = num / (den + 1e-16)
    return out + b_ref[...]


def _gat3_kernel(x_ref, adjt_ref, wl0, wr0, at0, b0, wl1, wr1, at1, b1,
                 wl2, wr2, at2, b2, out_ref):
    maskT = adjt_ref[0] > 0
    xb = x_ref[0]
    xb = _layer(xb, maskT, wl0, wr0, at0, b0)
    xb = _layer(xb, maskT, wl1, wr1, at1, b1)
    xb = _layer(xb, maskT, wl2, wr2, at2, b2)
    out_ref[0] = xb


@jax.jit
def kernel(batch_graph, adj, Wl0, Wr0, att0, b0, Wl1, Wr1, att1, b1,
           Wl2, Wr2, att2, b2):
    bsz, n, _ = batch_graph.shape
    dout = Wl2.shape[1]
    adjt = adj.transpose(0, 2, 1)

    def wspec(w):
        return pl.BlockSpec(w.shape, lambda b: (0,) * w.ndim)

    weights = [Wl0, Wr0, att0.reshape(1, -1), b0.reshape(1, -1),
               Wl1, Wr1, att1.reshape(1, -1), b1.reshape(1, -1),
               Wl2, Wr2, att2.reshape(1, -1), b2.reshape(1, -1)]

    out = pl.pallas_call(
        _gat3_kernel,
        grid=(bsz,),
        in_specs=[
            pl.BlockSpec((1, n, batch_graph.shape[2]), lambda b: (b, 0, 0)),
            pl.BlockSpec((1, n, n), lambda b: (b, 0, 0)),
        ] + [wspec(w) for w in weights],
        out_specs=pl.BlockSpec((1, n, dout), lambda b: (b, 0, 0)),
        out_shape=jax.ShapeDtypeStruct((bsz, n, dout), jnp.float32),
    )(batch_graph, adjt, *weights)
    return out
